# baseline (speedup 1.0000x reference)
import jax
import jax.numpy as jnp
from jax.experimental import pallas as pl
from jax.experimental.pallas import tpu as pltpu

RATIO = 0.5


def _topk_idx_kernel(p_ref, idx_ref):
    b = pl.program_id(0)
    C = p_ref.shape[2]
    K = idx_ref.shape[2]
    prow = p_ref[...].reshape(1, C)
    pj = jax.lax.broadcast_in_dim(prow, (C, C), (0, 1))
    pcol = prow.reshape(C, 1)
    pi = jax.lax.broadcast_in_dim(pcol, (C, C), (0, 1))
    gt = (pj > pi).astype(jnp.int32)
    jlt = (jax.lax.broadcasted_iota(jnp.int32, (C, C), 1)
           < jax.lax.broadcasted_iota(jnp.int32, (C, C), 0))
    eq = jnp.logical_and(pj == pi, jlt).astype(jnp.int32)
    rank = jnp.sum(gt + eq, axis=1, keepdims=True)
    rankb = jax.lax.broadcast_in_dim(rank, (C, K), (0, 1))
    r_iota = jax.lax.broadcasted_iota(jnp.int32, (C, K), 1)
    i_iota = jax.lax.broadcasted_iota(jnp.int32, (C, K), 0)
    onehot = (rankb == r_iota).astype(jnp.int32)
    idx = jnp.sum(onehot * i_iota, axis=0, keepdims=True) + b * C
    idx_ref[...] = idx.reshape(1, 1, K)


def _gather_kernel(idx_ref, x_ref, o_ref):
    del idx_ref
    o_ref[...] = x_ref[...]


def kernel(x, weight):
    B, C, H, W = x.shape
    K = int(RATIO * C)

    xr = x.reshape(B * C, 1, H, W)
    out = jax.lax.conv_general_dilated(
        xr, weight, window_strides=(1, 1), padding='VALID',
        dimension_numbers=('NCHW', 'OIHW', 'NCHW'))
    out = jnp.abs(out)
    p = jnp.sum(out, axis=-1)
    p = jnp.sum(p, axis=-1)
    p = p.reshape(B, C)

    flat_idx = pl.pallas_call(
        _topk_idx_kernel,
        grid=(B,),
        in_specs=[pl.BlockSpec((1, 1, C), lambda b: (b, 0, 0))],
        out_specs=pl.BlockSpec((1, 1, K), lambda b: (b, 0, 0)),
        out_shape=jax.ShapeDtypeStruct((B, 1, K), jnp.int32),
    )(p.reshape(B, 1, C))
    flat_idx = flat_idx.reshape(B * K)

    xf = x.reshape(B * C, H, W)
    grid_spec = pltpu.PrefetchScalarGridSpec(
        num_scalar_prefetch=1,
        grid=(B * K,),
        in_specs=[pl.BlockSpec((1, H, W), lambda i, idx: (idx[i], 0, 0))],
        out_specs=pl.BlockSpec((1, H, W), lambda i, idx: (i, 0, 0)),
    )
    sel = pl.pallas_call(
        _gather_kernel,
        grid_spec=grid_spec,
        out_shape=jax.ShapeDtypeStruct((B * K, H, W), x.dtype),
    )(flat_idx, xf)
    return sel.reshape(B, K, H, W)


# speedup vs baseline: 2.1146x; 2.1146x over previous
import functools

import jax
import jax.numpy as jnp
from jax.experimental import pallas as pl
from jax.experimental.pallas import tpu as pltpu

RATIO = 0.5

_TH_N, _TW_N = 6, 3
_WIN_H, _WIN_W = 37, 74
_TH_PER_STEP = 3
_LANES = 128


def _score_kernel(x_ref, w_ref, o_ref):
    f32 = jnp.float32
    wtaps = [[w_ref[0, 0, di, dj].astype(jnp.bfloat16).astype(f32)
              for dj in range(3)] for di in range(3)]

    def body(h, accs):
        new_accs = list(accs)
        for thl in range(_TH_PER_STEP):
            rows = [
                x_ref[0, thl, pl.ds(h + di, 1)]
                .reshape(224, _LANES).astype(f32)
                for di in range(3)
            ]
            conv = None
            for di in range(3):
                for dj in range(3):
                    t = rows[di][dj:dj + 222, :] * wtaps[di][dj]
                    conv = t if conv is None else conv + t
            a = jnp.abs(conv)
            for tw in range(_TW_N):
                acc = new_accs[thl * _TW_N + tw]
                base = tw * _WIN_W
                for wd in range(_WIN_W):
                    acc = acc + a[base + wd:base + wd + 1, :]
                new_accs[thl * _TW_N + tw] = acc
        return tuple(new_accs)

    init = tuple(jnp.zeros((1, _LANES), f32)
                 for _ in range(_TH_PER_STEP * _TW_N))
    accs = jax.lax.fori_loop(0, _WIN_H, body, init)
    o_ref[...] = jnp.concatenate(accs, axis=0).reshape(
        1, 1, _TH_PER_STEP * _TW_N, _LANES)


def _topk_idx_kernel(ws_ref, idx_ref):
    b = pl.program_id(0)
    C = ws_ref.shape[2]
    K = idx_ref.shape[2]
    ws = ws_ref[0]
    p = ws[0:1, :]
    for t in range(1, _TH_N * _TW_N):
        p = p + ws[t:t + 1, :]
    prow = p
    pj = jax.lax.broadcast_in_dim(prow, (C, C), (0, 1))
    pcol = prow.reshape(C, 1)
    pi = jax.lax.broadcast_in_dim(pcol, (C, C), (0, 1))
    gt = (pj > pi).astype(jnp.int32)
    jlt = (jax.lax.broadcasted_iota(jnp.int32, (C, C), 1)
           < jax.lax.broadcasted_iota(jnp.int32, (C, C), 0))
    eq = jnp.logical_and(pj == pi, jlt).astype(jnp.int32)
    rank = jnp.sum(gt + eq, axis=1, keepdims=True)
    rankb = jax.lax.broadcast_in_dim(rank, (C, K), (0, 1))
    r_iota = jax.lax.broadcasted_iota(jnp.int32, (C, K), 1)
    i_iota = jax.lax.broadcasted_iota(jnp.int32, (C, K), 0)
    onehot = (rankb == r_iota).astype(jnp.int32)
    idx = jnp.sum(onehot * i_iota, axis=0, keepdims=True) + b * C
    idx_ref[...] = idx.reshape(1, 1, K)


def _gather_kernel(idx_ref, x_ref, o_ref):
    del idx_ref
    o_ref[...] = x_ref[...]


def kernel(x, weight):
    B, C, H, W = x.shape
    K = int(RATIO * C)
    CG = C // _LANES
    TSTEPS = _TH_N // _TH_PER_STEP

    xs = jnp.stack([x[:, :, i * _WIN_H:i * _WIN_H + 39, :]
                    for i in range(_TH_N)], axis=1)
    xt = jnp.transpose(xs, (0, 1, 3, 4, 2)).astype(jnp.bfloat16)

    wsums = pl.pallas_call(
        _score_kernel,
        grid=(B, TSTEPS, CG),
        in_specs=[
            pl.BlockSpec((1, _TH_PER_STEP, 39, 224, _LANES),
                         lambda b, t, c: (b, t, 0, 0, c)),
            pl.BlockSpec((1, 1, 3, 3), lambda b, t, c: (0, 0, 0, 0)),
        ],
        out_specs=pl.BlockSpec((1, 1, _TH_PER_STEP * _TW_N, _LANES),
                               lambda b, t, c: (b, t, 0, c)),
        out_shape=jax.ShapeDtypeStruct(
            (B, TSTEPS, _TH_PER_STEP * _TW_N, C), jnp.float32),
    )(xt, weight)
    wsums = wsums.reshape(B, _TH_N * _TW_N, C)

    flat_idx = pl.pallas_call(
        _topk_idx_kernel,
        grid=(B,),
        in_specs=[pl.BlockSpec((1, _TH_N * _TW_N, C), lambda b: (b, 0, 0))],
        out_specs=pl.BlockSpec((1, 1, K), lambda b: (b, 0, 0)),
        out_shape=jax.ShapeDtypeStruct((B, 1, K), jnp.int32),
    )(wsums)
    flat_idx = flat_idx.reshape(B * K)

    xf = x.reshape(B * C, H, W)
    grid_spec = pltpu.PrefetchScalarGridSpec(
        num_scalar_prefetch=1,
        grid=(B * K,),
        in_specs=[pl.BlockSpec((1, H, W), lambda i, idx: (idx[i], 0, 0))],
        out_specs=pl.BlockSpec((1, H, W), lambda i, idx: (i, 0, 0)),
    )
    sel = pl.pallas_call(
        _gather_kernel,
        grid_spec=grid_spec,
        out_shape=jax.ShapeDtypeStruct((B * K, H, W), x.dtype),
    )(flat_idx, xf)
    return sel.reshape(B, K, H, W)


# speedup vs baseline: 2.2977x; 1.0866x over previous
import jax
import jax.numpy as jnp
from jax.experimental import pallas as pl
from jax.experimental.pallas import tpu as pltpu

RATIO = 0.5

_TH_N, _TW_N = 6, 3
_WIN_H, _WIN_W = 37, 74
_LANES = 128


def _score_kernel(x_ref, w_ref, o_ref):
    f32 = jnp.float32
    wtaps = [[w_ref[0, 0, di, dj].astype(jnp.bfloat16).astype(f32)
              for dj in range(3)] for di in range(3)]

    def body(h, accs):
        new_accs = list(accs)
        for th in range(_TH_N):
            rows = [
                x_ref[0, pl.ds(th * _WIN_H + h + di, 1)]
                .reshape(224, _LANES).astype(f32)
                for di in range(3)
            ]
            conv = None
            for di in range(3):
                for dj in range(3):
                    t = rows[di][dj:dj + 222, :] * wtaps[di][dj]
                    conv = t if conv is None else conv + t
            a = jnp.abs(conv)
            for tw in range(_TW_N):
                acc = new_accs[th * _TW_N + tw]
                base = tw * _WIN_W
                for wd in range(_WIN_W):
                    acc = acc + a[base + wd:base + wd + 1, :]
                new_accs[th * _TW_N + tw] = acc
        return tuple(new_accs)

    init = tuple(jnp.zeros((1, _LANES), f32) for _ in range(_TH_N * _TW_N))
    accs = jax.lax.fori_loop(0, _WIN_H, body, init)
    o_ref[...] = jnp.concatenate(accs, axis=0).reshape(
        1, 1, _TH_N * _TW_N, _LANES)


def _topk_idx_kernel(ws_ref, idx_ref):
    b = pl.program_id(0)
    C = ws_ref.shape[2]
    K = idx_ref.shape[2]
    ws = ws_ref[0]
    p = ws[0:1, :]
    for t in range(1, _TH_N * _TW_N):
        p = p + ws[t:t + 1, :]
    prow = p
    pj = jax.lax.broadcast_in_dim(prow, (C, C), (0, 1))
    pcol = prow.reshape(C, 1)
    pi = jax.lax.broadcast_in_dim(pcol, (C, C), (0, 1))
    gt = (pj > pi).astype(jnp.int32)
    jlt = (jax.lax.broadcasted_iota(jnp.int32, (C, C), 1)
           < jax.lax.broadcasted_iota(jnp.int32, (C, C), 0))
    eq = jnp.logical_and(pj == pi, jlt).astype(jnp.int32)
    rank = jnp.sum(gt + eq, axis=1, keepdims=True)
    rankb = jax.lax.broadcast_in_dim(rank, (C, K), (0, 1))
    r_iota = jax.lax.broadcasted_iota(jnp.int32, (C, K), 1)
    i_iota = jax.lax.broadcasted_iota(jnp.int32, (C, K), 0)
    onehot = (rankb == r_iota).astype(jnp.int32)
    idx = jnp.sum(onehot * i_iota, axis=0, keepdims=True) + b * C
    idx_ref[...] = idx.reshape(1, 1, K)


def _gather_kernel(idx_ref, x_ref, o_ref):
    del idx_ref
    o_ref[...] = x_ref[...]


def kernel(x, weight):
    B, C, H, W = x.shape
    K = int(RATIO * C)
    CG = C // _LANES

    xt = jnp.transpose(x, (0, 2, 3, 1)).astype(jnp.bfloat16)

    wsums = pl.pallas_call(
        _score_kernel,
        grid=(B, CG),
        in_specs=[
            pl.BlockSpec((1, H, W, _LANES), lambda b, c: (b, 0, 0, c)),
            pl.BlockSpec((1, 1, 3, 3), lambda b, c: (0, 0, 0, 0)),
        ],
        out_specs=pl.BlockSpec((1, 1, _TH_N * _TW_N, _LANES),
                               lambda b, c: (b, c, 0, 0)),
        out_shape=jax.ShapeDtypeStruct(
            (B, CG, _TH_N * _TW_N, _LANES), jnp.float32),
    )(xt, weight)
    wsums = jnp.transpose(wsums, (0, 2, 1, 3)).reshape(B, _TH_N * _TW_N, C)

    flat_idx = pl.pallas_call(
        _topk_idx_kernel,
        grid=(B,),
        in_specs=[pl.BlockSpec((1, _TH_N * _TW_N, C), lambda b: (b, 0, 0))],
        out_specs=pl.BlockSpec((1, 1, K), lambda b: (b, 0, 0)),
        out_shape=jax.ShapeDtypeStruct((B, 1, K), jnp.int32),
    )(wsums)
    flat_idx = flat_idx.reshape(B * K)

    xf = x.reshape(B * C, H, W)
    grid_spec = pltpu.PrefetchScalarGridSpec(
        num_scalar_prefetch=1,
        grid=(B * K,),
        in_specs=[pl.BlockSpec((1, H, W), lambda i, idx: (idx[i], 0, 0))],
        out_specs=pl.BlockSpec((1, H, W), lambda i, idx: (i, 0, 0)),
    )
    sel = pl.pallas_call(
        _gather_kernel,
        grid_spec=grid_spec,
        out_shape=jax.ShapeDtypeStruct((B * K, H, W), x.dtype),
    )(flat_idx, xf)
    return sel.reshape(B, K, H, W)
